# baseline (speedup 1.0000x reference)
import functools

import jax
import jax.numpy as jnp
from jax import lax
from jax.experimental import pallas as pl
from jax.experimental.pallas import tpu as pltpu
from jax.experimental.pallas import tpu_sc as plsc

_F32 = jnp.float32
_NTILE = 16
_NCORE = 2
_NW = _NTILE * _NCORE
_LN = 128


def _zero_vmem(buf, n_rows, d):
    zero16 = jnp.zeros((16,), _F32)
    for r in range(n_rows):
        for j in range(d // 16):
            buf[r, pl.ds(16 * j, 16)] = zero16


@functools.lru_cache(maxsize=None)
def _make_sc_agg(e_pad: int, n_acc: int, d: int):
    n_st = e_pad // _LN // _NW
    ph_st = min(n_st, 40)
    n_ph = n_st // ph_st
    acc_rows = n_acc // _NTILE
    nz = acc_rows // 16

    mesh = plsc.VectorSubcoreMesh(core_axis_name="c", subcore_axis_name="s")

    def body(g_hbm, src_hbm, dst_hbm, out_hbm, sidx, didx, rows, zbuf, acc,
             sg0, sg1, ss0, ss1, sz):
        c = lax.axis_index("c")
        s = lax.axis_index("s")
        wid = s * _NCORE + c
        _zero_vmem(zbuf, 16, d)
        zcs = [pltpu.async_copy(zbuf,
                                acc.at[pl.ds(s * acc_rows + i * 16, 16)], sz)
               for i in range(nz)]
        for zc in zcs:
            zc.wait()
        plsc.subcore_barrier()

        def slot(b):
            return rows.at[pl.ds(b * _LN, _LN)]

        for ph in range(n_ph):
            base = wid * n_st + ph * ph_st
            pltpu.sync_copy(src_hbm.at[pl.ds(base, ph_st)], sidx)
            pltpu.sync_copy(dst_hbm.at[pl.ds(base, ph_st)], didx)

            def step(it, _):
                q = it * 2
                g0 = pltpu.async_copy(g_hbm.at[sidx.at[q]], slot(0), sg0)
                g1 = pltpu.async_copy(g_hbm.at[sidx.at[q + 1]], slot(1), sg1)
                g0.wait()
                s0 = pltpu.async_copy(slot(0), acc.at[didx.at[q]], ss0,
                                      add=True)
                g1.wait()
                s1 = pltpu.async_copy(slot(1), acc.at[didx.at[q + 1]], ss1,
                                      add=True)
                s0.wait()
                s1.wait()
                return 0

            lax.fori_loop(0, ph_st // 2, step, 0)

        plsc.subcore_barrier()
        pltpu.sync_copy(acc.at[pl.ds(s * acc_rows, acc_rows)],
                        out_hbm.at[pl.ds(c * n_acc + s * acc_rows, acc_rows)])

    return pl.kernel(
        body,
        mesh=mesh,
        out_type=jax.ShapeDtypeStruct((2 * n_acc, d), _F32),
        scratch_types=[
            pltpu.VMEM((ph_st, _LN), jnp.int32),
            pltpu.VMEM((ph_st, _LN), jnp.int32),
            pltpu.VMEM((2 * _LN, d), _F32),
            pltpu.VMEM((16, d), _F32),
            pltpu.VMEM_SHARED((n_acc, d), _F32),
            pltpu.SemaphoreType.DMA,
            pltpu.SemaphoreType.DMA,
            pltpu.SemaphoreType.DMA,
            pltpu.SemaphoreType.DMA,
            pltpu.SemaphoreType.DMA,
        ],
    )


@functools.lru_cache(maxsize=None)
def _make_sc_egather(e_pad: int, d: int):
    n_st = e_pad // _LN // _NW

    mesh = plsc.VectorSubcoreMesh(core_axis_name="c", subcore_axis_name="s")

    def body(a_hbm, b_hbm, src_hbm, dst_hbm, oa_hbm, ob_hbm,
             sidx, didx, rows_a, rows_b, sg0, sg1):
        c = lax.axis_index("c")
        s = lax.axis_index("s")
        wid = s * _NCORE + c
        base = wid * n_st
        pltpu.sync_copy(src_hbm.at[pl.ds(base, n_st)], sidx)
        pltpu.sync_copy(dst_hbm.at[pl.ds(base, n_st)], didx)

        def step(it, _):
            q = it * 2
            ga0 = pltpu.async_copy(a_hbm.at[sidx.at[q]],
                                   rows_a.at[pl.ds(0, _LN)], sg0)
            gb0 = pltpu.async_copy(b_hbm.at[didx.at[q]],
                                   rows_b.at[pl.ds(0, _LN)], sg0)
            ga1 = pltpu.async_copy(a_hbm.at[sidx.at[q + 1]],
                                   rows_a.at[pl.ds(_LN, _LN)], sg1)
            gb1 = pltpu.async_copy(b_hbm.at[didx.at[q + 1]],
                                   rows_b.at[pl.ds(_LN, _LN)], sg1)
            ro = (base + q) * _LN
            ga0.wait()
            gb0.wait()
            pltpu.sync_copy(rows_a.at[pl.ds(0, _LN)], oa_hbm.at[pl.ds(ro, _LN)])
            pltpu.sync_copy(rows_b.at[pl.ds(0, _LN)], ob_hbm.at[pl.ds(ro, _LN)])
            ga1.wait()
            gb1.wait()
            pltpu.sync_copy(rows_a.at[pl.ds(_LN, _LN)],
                            oa_hbm.at[pl.ds(ro + _LN, _LN)])
            pltpu.sync_copy(rows_b.at[pl.ds(_LN, _LN)],
                            ob_hbm.at[pl.ds(ro + _LN, _LN)])
            return 0

        lax.fori_loop(0, n_st // 2, step, 0)

    return pl.kernel(
        body,
        mesh=mesh,
        out_type=(jax.ShapeDtypeStruct((e_pad, d), _F32),
                  jax.ShapeDtypeStruct((e_pad, d), _F32)),
        scratch_types=[
            pltpu.VMEM((n_st, _LN), jnp.int32),
            pltpu.VMEM((n_st, _LN), jnp.int32),
            pltpu.VMEM((2 * _LN, d), _F32),
            pltpu.VMEM((2 * _LN, d), _F32),
            pltpu.SemaphoreType.DMA,
            pltpu.SemaphoreType.DMA,
        ],
    )


def _pad_edges(src, dst, e_pad, dummy_dst):
    e = src.shape[0]
    pad = e_pad - e
    src_p = jnp.concatenate([src, jnp.zeros((pad,), jnp.int32)])
    dst_p = jnp.concatenate([dst, jnp.full((pad,), dummy_dst, jnp.int32)])
    return src_p.reshape(e_pad // _LN, _LN), dst_p.reshape(e_pad // _LN, _LN)


_RB = 1000
_EB = 512


def _dot(a, b):
    return jnp.dot(a, b, preferred_element_type=_F32)


def _linear_body(x_ref, w_ref, b_ref, o_ref):
    o_ref[...] = _dot(x_ref[...], w_ref[...]) + b_ref[...]


def _tc_linear(x, w, b):
    n, k = x.shape
    m = w.shape[1]
    return pl.pallas_call(
        _linear_body,
        grid=(pl.cdiv(n, _RB),),
        in_specs=[pl.BlockSpec((_RB, k), lambda i: (i, 0)),
                  pl.BlockSpec((k, m), lambda i: (0, 0)),
                  pl.BlockSpec((1, m), lambda i: (0, 0))],
        out_specs=pl.BlockSpec((_RB, m), lambda i: (i, 0)),
        out_shape=jax.ShapeDtypeStruct((n, m), _F32),
    )(x, w, b.reshape(1, -1))


def _sage_layer_body(h_ref, p_ref, c_ref, wl_ref, wr_ref, bl_ref, g_ref, b_ref,
                     o_ref):
    cnt = (jnp.sum(c_ref[0], axis=1, keepdims=True)
           + jnp.sum(c_ref[1], axis=1, keepdims=True))
    mean = (p_ref[0] + p_ref[1]) / jnp.maximum(cnt, 1.0)
    t = _dot(mean, wl_ref[...]) + bl_ref[...] + _dot(h_ref[...], wr_ref[...])
    t = jnp.maximum(t, 0.0)
    s = t + h_ref[...]
    mu = jnp.mean(s, axis=1, keepdims=True)
    var = jnp.mean((s - mu) ** 2, axis=1, keepdims=True)
    o_ref[...] = (s - mu) / jnp.sqrt(var + 1e-5) * g_ref[...] + b_ref[...]


def _tc_sage_layer(h, p, cnt, wl, bl, wr, g, b):
    n, d = h.shape
    n_acc = p.shape[0] // 2
    p3 = p.reshape(2, n_acc, d)
    c3 = cnt.reshape(2, n_acc, _LN)
    return pl.pallas_call(
        _sage_layer_body,
        grid=(n // _RB,),
        in_specs=[pl.BlockSpec((_RB, d), lambda i: (i, 0)),
                  pl.BlockSpec((2, _RB, d), lambda i: (0, i, 0)),
                  pl.BlockSpec((2, _RB, _LN), lambda i: (0, i, 0)),
                  pl.BlockSpec((d, d), lambda i: (0, 0)),
                  pl.BlockSpec((d, d), lambda i: (0, 0)),
                  pl.BlockSpec((1, d), lambda i: (0, 0)),
                  pl.BlockSpec((1, d), lambda i: (0, 0)),
                  pl.BlockSpec((1, d), lambda i: (0, 0))],
        out_specs=pl.BlockSpec((_RB, d), lambda i: (i, 0)),
        out_shape=jax.ShapeDtypeStruct((n, d), _F32),
    )(h, p3, c3, wl, wr, bl.reshape(1, -1), g.reshape(1, -1), b.reshape(1, -1))


def _mlp_body(x_ref, w1_ref, b1_ref, w2_ref, b2_ref, w3_ref, b3_ref, o_ref):
    h = jnp.maximum(_dot(x_ref[...], w1_ref[...]) + b1_ref[...], 0.0)
    h = jnp.maximum(_dot(h, w2_ref[...]) + b2_ref[...], 0.0)
    o_ref[...] = _dot(h, w3_ref[...]) + b3_ref[...]


def _tc_mlp(x, p):
    n, d = x.shape
    w3p = jnp.pad(p['W3'], ((0, 0), (0, 7)))
    b3p = jnp.pad(p['b3'].reshape(1, 1), ((0, 0), (0, 7)))
    return pl.pallas_call(
        _mlp_body,
        grid=(pl.cdiv(n, _RB),),
        in_specs=[pl.BlockSpec((_RB, d), lambda i: (i, 0)),
                  pl.BlockSpec((d, d), lambda i: (0, 0)),
                  pl.BlockSpec((1, d), lambda i: (0, 0)),
                  pl.BlockSpec((d, d), lambda i: (0, 0)),
                  pl.BlockSpec((1, d), lambda i: (0, 0)),
                  pl.BlockSpec((d, 8), lambda i: (0, 0)),
                  pl.BlockSpec((1, 8), lambda i: (0, 0))],
        out_specs=pl.BlockSpec((_RB, 8), lambda i: (i, 0)),
        out_shape=jax.ShapeDtypeStruct((n, 8), _F32),
    )(x, p['W1'], p['b1'].reshape(1, -1), p['W2'], p['b2'].reshape(1, -1),
      w3p, b3p)[:, 0]


def _edge_head_body(ga_ref, gb_ref, xe_ref, ew_ref, eb_ref, w1c_ref, b1_ref,
                    w2_ref, b2_ref, w3_ref, b3_ref, o_ref):
    he = jnp.maximum(_dot(xe_ref[...], ew_ref[...]) + eb_ref[...], 0.0)
    t = jnp.maximum(ga_ref[...] + gb_ref[...] + _dot(he, w1c_ref[...])
                    + b1_ref[...], 0.0)
    t = jnp.maximum(_dot(t, w2_ref[...]) + b2_ref[...], 0.0)
    o_ref[...] = _dot(t, w3_ref[...]) + b3_ref[...]


def _tc_edge_head(ga, gb, x_edge, ew, eb, w1c, hp):
    e, d = x_edge.shape[0], ga.shape[1]
    w3p = jnp.pad(hp['W3'], ((0, 0), (0, 7)))
    b3p = jnp.pad(hp['b3'].reshape(1, 1), ((0, 0), (0, 7)))
    return pl.pallas_call(
        _edge_head_body,
        grid=(pl.cdiv(e, _EB),),
        in_specs=[pl.BlockSpec((_EB, d), lambda i: (i, 0)),
                  pl.BlockSpec((_EB, d), lambda i: (i, 0)),
                  pl.BlockSpec((_EB, 16), lambda i: (i, 0)),
                  pl.BlockSpec((16, d), lambda i: (0, 0)),
                  pl.BlockSpec((1, d), lambda i: (0, 0)),
                  pl.BlockSpec((d, d), lambda i: (0, 0)),
                  pl.BlockSpec((1, d), lambda i: (0, 0)),
                  pl.BlockSpec((d, d), lambda i: (0, 0)),
                  pl.BlockSpec((1, d), lambda i: (0, 0)),
                  pl.BlockSpec((d, 8), lambda i: (0, 0)),
                  pl.BlockSpec((1, 8), lambda i: (0, 0))],
        out_specs=pl.BlockSpec((_EB, 8), lambda i: (i, 0)),
        out_shape=jax.ShapeDtypeStruct((e, 8), _F32),
    )(ga, gb, x_edge, ew, eb.reshape(1, -1), w1c,
      hp['b1'].reshape(1, -1), hp['W2'], hp['b2'].reshape(1, -1), w3p, b3p)[:, 0]


def _round_up(x, m):
    return (x + m - 1) // m * m


def _sage_stack(x, edge_index, p):
    n, d = x.shape
    e = edge_index.shape[1]
    n_acc = _round_up(n + 1, _NTILE * 64)
    e_pad = _round_up(e, _NW * 2 * _LN)
    src2, dst2 = _pad_edges(edge_index[0], edge_index[1], e_pad, n)
    ones_g = jnp.full((n, d), 1.0 / 128.0, _F32)
    cnt = _make_sc_agg(e_pad, n_acc, d)(ones_g, src2, dst2)
    h = _tc_linear(x, p['in_W'], p['in_b'])
    for lp in p['layers']:
        part = _make_sc_agg(e_pad, n_acc, d)(h, src2, dst2)
        h = _tc_sage_layer(h, part, cnt, lp['Wl'], lp['bl'], lp['Wr'],
                           lp['g'], lp['b'])
    return h, src2, dst2, e_pad


def kernel(x2d, edge_index_2d, x1d, edge_index_1d, x_edge, params):
    p = params
    h2, _, _, _ = _sage_stack(x2d, edge_index_2d, p['gnn2d'])
    d2 = _tc_mlp(h2, p['head2d'])

    h1, src2, dst2, e_pad = _sage_stack(x1d, edge_index_1d, p['gnn1d'])
    d1 = _tc_mlp(h1, p['head1d'])
    inlet = _tc_mlp(h1, p['head_inlet'])

    w1 = p['head_edge']['W1']
    ab = _tc_linear(h1, jnp.concatenate([w1[:128], w1[128:256]], axis=1),
                    jnp.zeros((256,), _F32))
    a_rows = ab[:, :128]
    b_rows = ab[:, 128:]
    ga, gb = _make_sc_egather(e_pad, 128)(a_rows, b_rows, src2, dst2)
    eflow = _tc_edge_head(ga, gb, x_edge, p['edge_W'], p['edge_b'],
                          w1[256:384], p['head_edge'])
    return (d2, d1, inlet, eflow)


# speedup vs baseline: 1.1853x; 1.1853x over previous
import functools

import jax
import jax.numpy as jnp
from jax import lax
from jax.experimental import pallas as pl
from jax.experimental.pallas import tpu as pltpu
from jax.experimental.pallas import tpu_sc as plsc

_F32 = jnp.float32
_NTILE = 16
_NCORE = 2
_NW = _NTILE * _NCORE
_LN = 128


def _zero_vmem(buf, n_rows, d):
    zero16 = jnp.zeros((16,), _F32)
    for r in range(n_rows):
        for j in range(d // 16):
            buf[r, pl.ds(16 * j, 16)] = zero16


@functools.lru_cache(maxsize=None)
def _make_sc_agg(e_pad: int, n_acc: int, d: int):
    n_st = e_pad // _LN // _NW
    ph_st = min(n_st, 40)
    n_ph = n_st // ph_st
    acc_rows = n_acc // _NTILE
    nz = acc_rows // 16

    mesh = plsc.VectorSubcoreMesh(core_axis_name="c", subcore_axis_name="s")

    def body(g_hbm, src_hbm, dst_hbm, out_hbm, sidx, didx, rows, zbuf, acc,
             sg0, sg1, ss0, ss1, sz):
        c = lax.axis_index("c")
        s = lax.axis_index("s")
        wid = s * _NCORE + c
        _zero_vmem(zbuf, 16, d)
        zcs = [pltpu.async_copy(zbuf,
                                acc.at[pl.ds(s * acc_rows + i * 16, 16)], sz)
               for i in range(nz)]
        for zc in zcs:
            zc.wait()
        plsc.subcore_barrier()

        def slot(b):
            return rows.at[pl.ds(b * _LN, _LN)]

        for ph in range(n_ph):
            base = wid * n_st + ph * ph_st
            pltpu.sync_copy(src_hbm.at[pl.ds(base, ph_st)], sidx)
            pltpu.sync_copy(dst_hbm.at[pl.ds(base, ph_st)], didx)

            def step(it, _):
                q = it * 2
                g0 = pltpu.async_copy(g_hbm.at[sidx.at[q]], slot(0), sg0)
                g1 = pltpu.async_copy(g_hbm.at[sidx.at[q + 1]], slot(1), sg1)
                g0.wait()
                s0 = pltpu.async_copy(slot(0), acc.at[didx.at[q]], ss0,
                                      add=True)
                g1.wait()
                s1 = pltpu.async_copy(slot(1), acc.at[didx.at[q + 1]], ss1,
                                      add=True)
                s0.wait()
                s1.wait()
                return 0

            lax.fori_loop(0, ph_st // 2, step, 0)

        plsc.subcore_barrier()
        pltpu.sync_copy(acc.at[pl.ds(s * acc_rows, acc_rows)],
                        out_hbm.at[pl.ds(c * n_acc + s * acc_rows, acc_rows)])

    return pl.kernel(
        body,
        mesh=mesh,
        out_type=jax.ShapeDtypeStruct((2 * n_acc, d), _F32),
        scratch_types=[
            pltpu.VMEM((ph_st, _LN), jnp.int32),
            pltpu.VMEM((ph_st, _LN), jnp.int32),
            pltpu.VMEM((2 * _LN, d), _F32),
            pltpu.VMEM((16, d), _F32),
            pltpu.VMEM_SHARED((n_acc, d), _F32),
            pltpu.SemaphoreType.DMA,
            pltpu.SemaphoreType.DMA,
            pltpu.SemaphoreType.DMA,
            pltpu.SemaphoreType.DMA,
            pltpu.SemaphoreType.DMA,
        ],
    )


@functools.lru_cache(maxsize=None)
def _make_sc_cnt(e_pad: int, n_acc: int):
    n_st = e_pad // _LN // _NW
    acc_rows = n_acc // _NTILE
    nz = acc_rows // 16

    mesh = plsc.VectorSubcoreMesh(core_axis_name="c", subcore_axis_name="s")

    def body(dst_hbm, out_hbm, didx, ones_v, zbuf, cnt, s0, s1, s2, s3, sz):
        c = lax.axis_index("c")
        s = lax.axis_index("s")
        wid = s * _NCORE + c
        frac16 = jnp.full((16,), 1.0 / 128.0, _F32)
        _zero_vmem(zbuf, 16, _LN)
        for r in range(_LN):
            for j in range(8):
                ones_v[r, pl.ds(16 * j, 16)] = frac16
        pltpu.sync_copy(dst_hbm.at[pl.ds(wid * n_st, n_st)], didx)
        zcs = [pltpu.async_copy(zbuf,
                                cnt.at[pl.ds(s * acc_rows + i * 16, 16)], sz)
               for i in range(nz)]
        for zc in zcs:
            zc.wait()
        plsc.subcore_barrier()

        sems = (s0, s1, s2, s3)

        def step(it, _):
            q = it * 4
            cps = [pltpu.async_copy(ones_v, cnt.at[didx.at[q + j]], sems[j],
                                    add=True) for j in range(4)]
            for cp in cps:
                cp.wait()
            return 0

        lax.fori_loop(0, n_st // 4, step, 0)
        plsc.subcore_barrier()
        pltpu.sync_copy(cnt.at[pl.ds(s * acc_rows, acc_rows)],
                        out_hbm.at[pl.ds(c * n_acc + s * acc_rows, acc_rows)])

    return pl.kernel(
        body,
        mesh=mesh,
        out_type=jax.ShapeDtypeStruct((2 * n_acc, _LN), _F32),
        scratch_types=[
            pltpu.VMEM((n_st, _LN), jnp.int32),
            pltpu.VMEM((_LN, _LN), _F32),
            pltpu.VMEM((16, _LN), _F32),
            pltpu.VMEM_SHARED((n_acc, _LN), _F32),
            pltpu.SemaphoreType.DMA,
            pltpu.SemaphoreType.DMA,
            pltpu.SemaphoreType.DMA,
            pltpu.SemaphoreType.DMA,
            pltpu.SemaphoreType.DMA,
        ],
    )


@functools.lru_cache(maxsize=None)
def _make_sc_egather(e_pad: int, d: int):
    n_st = e_pad // _LN // _NW

    mesh = plsc.VectorSubcoreMesh(core_axis_name="c", subcore_axis_name="s")

    def body(a_hbm, b_hbm, src_hbm, dst_hbm, oa_hbm, ob_hbm,
             sidx, didx, rows_a, rows_b, sg0, sg1):
        c = lax.axis_index("c")
        s = lax.axis_index("s")
        wid = s * _NCORE + c
        base = wid * n_st
        pltpu.sync_copy(src_hbm.at[pl.ds(base, n_st)], sidx)
        pltpu.sync_copy(dst_hbm.at[pl.ds(base, n_st)], didx)

        def step(it, _):
            q = it * 2
            ga0 = pltpu.async_copy(a_hbm.at[sidx.at[q]],
                                   rows_a.at[pl.ds(0, _LN)], sg0)
            gb0 = pltpu.async_copy(b_hbm.at[didx.at[q]],
                                   rows_b.at[pl.ds(0, _LN)], sg0)
            ga1 = pltpu.async_copy(a_hbm.at[sidx.at[q + 1]],
                                   rows_a.at[pl.ds(_LN, _LN)], sg1)
            gb1 = pltpu.async_copy(b_hbm.at[didx.at[q + 1]],
                                   rows_b.at[pl.ds(_LN, _LN)], sg1)
            ro = (base + q) * _LN
            ga0.wait()
            gb0.wait()
            pltpu.sync_copy(rows_a.at[pl.ds(0, _LN)], oa_hbm.at[pl.ds(ro, _LN)])
            pltpu.sync_copy(rows_b.at[pl.ds(0, _LN)], ob_hbm.at[pl.ds(ro, _LN)])
            ga1.wait()
            gb1.wait()
            pltpu.sync_copy(rows_a.at[pl.ds(_LN, _LN)],
                            oa_hbm.at[pl.ds(ro + _LN, _LN)])
            pltpu.sync_copy(rows_b.at[pl.ds(_LN, _LN)],
                            ob_hbm.at[pl.ds(ro + _LN, _LN)])
            return 0

        lax.fori_loop(0, n_st // 2, step, 0)

    return pl.kernel(
        body,
        mesh=mesh,
        out_type=(jax.ShapeDtypeStruct((e_pad, d), _F32),
                  jax.ShapeDtypeStruct((e_pad, d), _F32)),
        scratch_types=[
            pltpu.VMEM((n_st, _LN), jnp.int32),
            pltpu.VMEM((n_st, _LN), jnp.int32),
            pltpu.VMEM((2 * _LN, d), _F32),
            pltpu.VMEM((2 * _LN, d), _F32),
            pltpu.SemaphoreType.DMA,
            pltpu.SemaphoreType.DMA,
        ],
    )


def _pad_edges(src, dst, e_pad, dummy_dst):
    e = src.shape[0]
    pad = e_pad - e
    src_p = jnp.concatenate([src, jnp.zeros((pad,), jnp.int32)])
    dst_p = jnp.concatenate([dst, jnp.full((pad,), dummy_dst, jnp.int32)])
    return src_p.reshape(e_pad // _LN, _LN), dst_p.reshape(e_pad // _LN, _LN)


_RB = 1000
_EB = 512


def _dot(a, b):
    return jnp.dot(a, b, preferred_element_type=_F32)


def _linear_body(x_ref, w_ref, b_ref, o_ref):
    o_ref[...] = _dot(x_ref[...], w_ref[...]) + b_ref[...]


def _tc_linear(x, w, b):
    n, k = x.shape
    m = w.shape[1]
    return pl.pallas_call(
        _linear_body,
        grid=(pl.cdiv(n, _RB),),
        in_specs=[pl.BlockSpec((_RB, k), lambda i: (i, 0)),
                  pl.BlockSpec((k, m), lambda i: (0, 0)),
                  pl.BlockSpec((1, m), lambda i: (0, 0))],
        out_specs=pl.BlockSpec((_RB, m), lambda i: (i, 0)),
        out_shape=jax.ShapeDtypeStruct((n, m), _F32),
    )(x, w, b.reshape(1, -1))


def _sage_layer_body(h_ref, p_ref, c_ref, wl_ref, wr_ref, bl_ref, g_ref, b_ref,
                     o_ref):
    cnt = (jnp.sum(c_ref[0], axis=1, keepdims=True)
           + jnp.sum(c_ref[1], axis=1, keepdims=True))
    mean = (p_ref[0] + p_ref[1]) / jnp.maximum(cnt, 1.0)
    t = _dot(mean, wl_ref[...]) + bl_ref[...] + _dot(h_ref[...], wr_ref[...])
    t = jnp.maximum(t, 0.0)
    s = t + h_ref[...]
    mu = jnp.mean(s, axis=1, keepdims=True)
    var = jnp.mean((s - mu) ** 2, axis=1, keepdims=True)
    o_ref[...] = (s - mu) / jnp.sqrt(var + 1e-5) * g_ref[...] + b_ref[...]


def _tc_sage_layer(h, p, cnt, wl, bl, wr, g, b):
    n, d = h.shape
    n_acc = p.shape[0] // 2
    p3 = p.reshape(2, n_acc, d)
    c3 = cnt.reshape(2, n_acc, _LN)
    return pl.pallas_call(
        _sage_layer_body,
        grid=(n // _RB,),
        in_specs=[pl.BlockSpec((_RB, d), lambda i: (i, 0)),
                  pl.BlockSpec((2, _RB, d), lambda i: (0, i, 0)),
                  pl.BlockSpec((2, _RB, _LN), lambda i: (0, i, 0)),
                  pl.BlockSpec((d, d), lambda i: (0, 0)),
                  pl.BlockSpec((d, d), lambda i: (0, 0)),
                  pl.BlockSpec((1, d), lambda i: (0, 0)),
                  pl.BlockSpec((1, d), lambda i: (0, 0)),
                  pl.BlockSpec((1, d), lambda i: (0, 0))],
        out_specs=pl.BlockSpec((_RB, d), lambda i: (i, 0)),
        out_shape=jax.ShapeDtypeStruct((n, d), _F32),
    )(h, p3, c3, wl, wr, bl.reshape(1, -1), g.reshape(1, -1), b.reshape(1, -1))


def _mlp_body(x_ref, w1_ref, b1_ref, w2_ref, b2_ref, w3_ref, b3_ref, o_ref):
    h = jnp.maximum(_dot(x_ref[...], w1_ref[...]) + b1_ref[...], 0.0)
    h = jnp.maximum(_dot(h, w2_ref[...]) + b2_ref[...], 0.0)
    o_ref[...] = _dot(h, w3_ref[...]) + b3_ref[...]


def _tc_mlp(x, p):
    n, d = x.shape
    w3p = jnp.pad(p['W3'], ((0, 0), (0, 7)))
    b3p = jnp.pad(p['b3'].reshape(1, 1), ((0, 0), (0, 7)))
    return pl.pallas_call(
        _mlp_body,
        grid=(pl.cdiv(n, _RB),),
        in_specs=[pl.BlockSpec((_RB, d), lambda i: (i, 0)),
                  pl.BlockSpec((d, d), lambda i: (0, 0)),
                  pl.BlockSpec((1, d), lambda i: (0, 0)),
                  pl.BlockSpec((d, d), lambda i: (0, 0)),
                  pl.BlockSpec((1, d), lambda i: (0, 0)),
                  pl.BlockSpec((d, 8), lambda i: (0, 0)),
                  pl.BlockSpec((1, 8), lambda i: (0, 0))],
        out_specs=pl.BlockSpec((_RB, 8), lambda i: (i, 0)),
        out_shape=jax.ShapeDtypeStruct((n, 8), _F32),
    )(x, p['W1'], p['b1'].reshape(1, -1), p['W2'], p['b2'].reshape(1, -1),
      w3p, b3p)[:, 0]


def _edge_head_body(ga_ref, gb_ref, xe_ref, ew_ref, eb_ref, w1c_ref, b1_ref,
                    w2_ref, b2_ref, w3_ref, b3_ref, o_ref):
    he = jnp.maximum(_dot(xe_ref[...], ew_ref[...]) + eb_ref[...], 0.0)
    t = jnp.maximum(ga_ref[...] + gb_ref[...] + _dot(he, w1c_ref[...])
                    + b1_ref[...], 0.0)
    t = jnp.maximum(_dot(t, w2_ref[...]) + b2_ref[...], 0.0)
    o_ref[...] = _dot(t, w3_ref[...]) + b3_ref[...]


def _tc_edge_head(ga, gb, x_edge, ew, eb, w1c, hp):
    e, d = x_edge.shape[0], ga.shape[1]
    w3p = jnp.pad(hp['W3'], ((0, 0), (0, 7)))
    b3p = jnp.pad(hp['b3'].reshape(1, 1), ((0, 0), (0, 7)))
    return pl.pallas_call(
        _edge_head_body,
        grid=(pl.cdiv(e, _EB),),
        in_specs=[pl.BlockSpec((_EB, d), lambda i: (i, 0)),
                  pl.BlockSpec((_EB, d), lambda i: (i, 0)),
                  pl.BlockSpec((_EB, 16), lambda i: (i, 0)),
                  pl.BlockSpec((16, d), lambda i: (0, 0)),
                  pl.BlockSpec((1, d), lambda i: (0, 0)),
                  pl.BlockSpec((d, d), lambda i: (0, 0)),
                  pl.BlockSpec((1, d), lambda i: (0, 0)),
                  pl.BlockSpec((d, d), lambda i: (0, 0)),
                  pl.BlockSpec((1, d), lambda i: (0, 0)),
                  pl.BlockSpec((d, 8), lambda i: (0, 0)),
                  pl.BlockSpec((1, 8), lambda i: (0, 0))],
        out_specs=pl.BlockSpec((_EB, 8), lambda i: (i, 0)),
        out_shape=jax.ShapeDtypeStruct((e, 8), _F32),
    )(ga, gb, x_edge, ew, eb.reshape(1, -1), w1c,
      hp['b1'].reshape(1, -1), hp['W2'], hp['b2'].reshape(1, -1), w3p, b3p)[:, 0]


def _round_up(x, m):
    return (x + m - 1) // m * m


def _sage_stack(x, edge_index, p):
    n, d = x.shape
    e = edge_index.shape[1]
    n_acc = _round_up(n + 1, _NTILE * 64)
    e_pad = _round_up(e, _NW * 2 * _LN)
    src2, dst2 = _pad_edges(edge_index[0], edge_index[1], e_pad, n)
    cnt = _make_sc_cnt(e_pad, n_acc)(dst2)
    h = _tc_linear(x, p['in_W'], p['in_b'])
    for lp in p['layers']:
        part = _make_sc_agg(e_pad, n_acc, d)(h, src2, dst2)
        h = _tc_sage_layer(h, part, cnt, lp['Wl'], lp['bl'], lp['Wr'],
                           lp['g'], lp['b'])
    return h, src2, dst2, e_pad


def kernel(x2d, edge_index_2d, x1d, edge_index_1d, x_edge, params):
    p = params
    h2, _, _, _ = _sage_stack(x2d, edge_index_2d, p['gnn2d'])
    d2 = _tc_mlp(h2, p['head2d'])

    h1, src2, dst2, e_pad = _sage_stack(x1d, edge_index_1d, p['gnn1d'])
    d1 = _tc_mlp(h1, p['head1d'])
    inlet = _tc_mlp(h1, p['head_inlet'])

    w1 = p['head_edge']['W1']
    ab = _tc_linear(h1, jnp.concatenate([w1[:128], w1[128:256]], axis=1),
                    jnp.zeros((256,), _F32))
    a_rows = ab[:, :128]
    b_rows = ab[:, 128:]
    ga, gb = _make_sc_egather(e_pad, 128)(a_rows, b_rows, src2, dst2)
    eflow = _tc_edge_head(ga, gb, x_edge, p['edge_W'], p['edge_b'],
                          w1[256:384], p['head_edge'])
    return (d2, d1, inlet, eflow)
